# baseline (speedup 1.0000x reference)
import functools

import jax
import jax.numpy as jnp
from jax import lax
from jax.experimental import pallas as pl
from jax.experimental.pallas import tpu as pltpu
from jax.experimental.pallas import tpu_sc as plsc

_B, _S = 128, 1024
_VOCAB, _DIM, _PAD, _BIO = 25, 256, 20, 5
_N = _B * _S
_T = 8192

_NW = 32
_NTOK = _N // _NW
_G = 32
_NSUB = _G // 16
_NGROUP = _NTOK // _G
_PHTOK = 1024
_GPP = _PHTOK // _G


def _fold_body(table_ref, wf_ref, wp_ref, bp_ref, bf_ref, w1_ref, w2_ref, b2_ref):
    tbl = table_ref[...]
    row = lax.broadcasted_iota(jnp.int32, tbl.shape, 0)
    tbl = jnp.where(row == _PAD, 0.0, tbl)
    wf_top = wf_ref[0:_DIM, :]
    wf_bot = wf_ref[_DIM:2 * _DIM, :]
    w1 = jnp.dot(tbl, wf_top, preferred_element_type=jnp.float32)
    w2 = jnp.dot(wp_ref[...], wf_bot, preferred_element_type=jnp.float32)
    b2 = jnp.dot(bp_ref[...], wf_bot, preferred_element_type=jnp.float32) + bf_ref[...]
    w1_ref[...] = w1 - jnp.mean(w1, axis=1, keepdims=True)
    w2_ref[...] = w2 - jnp.mean(w2, axis=1, keepdims=True)
    b2_ref[...] = b2 - jnp.mean(b2, axis=1, keepdims=True)


def _fold(table, Wp, bp, Wf, bf):
    table_pad = jnp.pad(table, ((0, 32 - _VOCAB), (0, 0)))
    wp_pad = jnp.pad(Wp, ((0, 8 - _BIO), (0, 0)))
    return pl.pallas_call(
        _fold_body,
        out_shape=(
            jax.ShapeDtypeStruct((32, _DIM), jnp.float32),
            jax.ShapeDtypeStruct((8, _DIM), jnp.float32),
            jax.ShapeDtypeStruct((1, _DIM), jnp.float32),
        ),
    )(table_pad, Wf, wp_pad, bp.reshape(1, _DIM), bf.reshape(1, _DIM))


def _tc_body(idx_ref, bio_ref, w1_ref, w2_ref, b2_ref, g_ref, bt_ref, out_ref):
    idx = idx_ref[...]
    oh = (idx == lax.broadcasted_iota(jnp.int32, (_T, 32), 1)).astype(jnp.float32)
    c = jnp.dot(oh, w1_ref[...], preferred_element_type=jnp.float32)
    c = c + jnp.dot(bio_ref[...], w2_ref[...], preferred_element_type=jnp.float32)
    c = c + b2_ref[...]
    v = jnp.mean(c * c, axis=1, keepdims=True)
    out_ref[...] = c * lax.rsqrt(v + 1e-5) * g_ref[...] + bt_ref[...]


def _sc_body(idx_hbm, bio_hbm, t2c_hbm, w2x_hbm, b2x_hbm, gx_hbm, btx_hbm,
             out_hbm,
             idx_v, bio_v, t2c_v, w2x_v, b2x_v, gx_v, btx_v, stage_v, outbuf_v,
             sem0, sem1):
    cid = lax.axis_index("c")
    sid = lax.axis_index("s")
    wid = sid * 2 + cid
    tok0 = wid * _NTOK

    pltpu.sync_copy(idx_hbm.at[pl.ds(tok0, _NTOK)], idx_v)
    pltpu.sync_copy(t2c_hbm, t2c_v)
    pltpu.sync_copy(w2x_hbm, w2x_v)
    pltpu.sync_copy(b2x_hbm, b2x_v)
    pltpu.sync_copy(gx_hbm, gx_v)
    pltpu.sync_copy(btx_hbm, btx_v)

    lane = lax.iota(jnp.int32, 16)
    sems = [sem0, sem1]

    def pair_body(p, carry):
      @pl.when(lax.rem(p, _GPP // 2) == 0)
      def _stage_bio():
          ph = lax.div(p, _GPP // 2)
          pltpu.sync_copy(
              bio_hbm.at[pl.ds((tok0 + ph * _PHTOK) * 8, _PHTOK * 8)], bio_v)

      for h in range(2):
        g = p * 2 + h
        base = g * _G
        lbase = lax.rem(g, _GPP) * _G
        rowaddr = []
        bvecs = []
        for s in range(_NSUB):
            idxv = idx_v[pl.ds(base + s * 16, 16)]
            rowaddr.append(idxv * 257)
            baddr = (lbase + s * 16) * 8 + lane * 8
            bvecs.append([plsc.load_gather(bio_v, [baddr + k]) for k in range(_BIO)])

        zero = jnp.zeros((16,), jnp.float32)

        def p1(i, accs):
            new = list(accs)
            for u in range(2):
                d = i + u
                w = [w2x_v[pl.ds(k * (_DIM * 16) + d * 16, 16)] for k in range(_BIO)]
                b2 = b2x_v[pl.ds(d * 16, 16)]
                for s in range(_NSUB):
                    g0 = plsc.load_gather(t2c_v, [rowaddr[s] + d]) + b2
                    m01 = bvecs[s][0] * w[0] + bvecs[s][1] * w[1]
                    m23 = bvecs[s][2] * w[2] + bvecs[s][3] * w[3]
                    e = (g0 + bvecs[s][4] * w[4]) + (m01 + m23)
                    stage_v[pl.ds(d * _G + s * 16, 16)] = e
                    j = u * _NSUB + s
                    new[j] = new[j] + e * e
            return tuple(new)

        accs = plsc.parallel_loop(
            0, _DIM, 2, unroll=8,
            carry=tuple(zero for _ in range(2 * _NSUB)))(p1)

        invs = []
        for s in range(_NSUB):
            x = (accs[s] + accs[_NSUB + s]) * (1.0 / _DIM) + 1e-5
            i32 = plsc.bitcast(x, jnp.int32)
            y = plsc.bitcast(jnp.int32(0x5F3759DF) - (i32 >> 1), jnp.float32)
            for _ in range(3):
                y = y * (1.5 - 0.5 * x * y * y)
            invs.append(y)

        rowidx = [h * _G + lane + s * 16 for s in range(_NSUB)]

        @pl.when(p > 0)
        def _wait_prev():
            pltpu.make_async_copy(
                outbuf_v.at[pl.ds(h * _G, _G), 0:_DIM],
                out_hbm.at[pl.ds(tok0 + (g - 2) * _G, _G), :],
                sems[h]).wait()

        def p2(d, c2):
            gsp = gx_v[pl.ds(d * 16, 16)]
            bsp = btx_v[pl.ds(d * 16, 16)]
            dcol = jnp.full((16,), d, jnp.int32)
            for s in range(_NSUB):
                e = stage_v[pl.ds(d * _G + s * 16, 16)]
                plsc.store_scatter(outbuf_v, [rowidx[s], dcol],
                                   (e * invs[s]) * gsp + bsp)
            return c2

        plsc.parallel_loop(0, _DIM, 1, unroll=8, carry=jnp.int32(0))(p2)
        pltpu.make_async_copy(outbuf_v.at[pl.ds(h * _G, _G), 0:_DIM],
                              out_hbm.at[pl.ds(tok0 + base, _G), :],
                              sems[h]).start()
      return carry

    lax.fori_loop(0, _NGROUP // 2, pair_body, 0)
    for h in range(2):
        g = _NGROUP - 2 + h
        pltpu.make_async_copy(outbuf_v.at[pl.ds(h * _G, _G), 0:_DIM],
                              out_hbm.at[pl.ds(tok0 + g * _G, _G), :],
                              sems[h]).wait()


def _sc_run(idx, bio8_flat, t2c_flat, w2x_flat, b2x_flat, gx_flat, btx_flat):
    mesh = plsc.VectorSubcoreMesh(core_axis_name="c", subcore_axis_name="s")
    f = functools.partial(
        pl.kernel, _sc_body,
        out_type=jax.ShapeDtypeStruct((_N, _DIM), jnp.float32),
        mesh=mesh,
        scratch_types=[
            pltpu.VMEM((_NTOK,), jnp.int32),
            pltpu.VMEM((_PHTOK * 8,), jnp.float32),
            pltpu.VMEM((32 * 257,), jnp.float32),
            pltpu.VMEM((_BIO * _DIM * 16,), jnp.float32),
            pltpu.VMEM((_DIM * 16,), jnp.float32),
            pltpu.VMEM((_DIM * 16,), jnp.float32),
            pltpu.VMEM((_DIM * 16,), jnp.float32),
            pltpu.VMEM((_DIM * _G,), jnp.float32),
            pltpu.VMEM((2 * _G, 257), jnp.float32),
            pltpu.SemaphoreType.DMA,
            pltpu.SemaphoreType.DMA,
        ],
        compiler_params=pltpu.CompilerParams(needs_layout_passes=False),
    )()
    return f(idx, bio8_flat, t2c_flat, w2x_flat, b2x_flat, gx_flat, btx_flat)


def kernel(aa_indices, biochem_features, table, Wp, bp, Wf, bf, gamma, beta):
    idx = aa_indices.astype(jnp.int32).reshape(_N)
    bio = jnp.pad(biochem_features.reshape(_N, _BIO), ((0, 0), (0, 8 - _BIO)))
    w1c, w2c, b2c = _fold(table, Wp, bp, Wf, bf)

    t2c_flat = jnp.pad(w1c, ((0, 0), (0, 1))).reshape(-1)
    w2x_flat = jnp.broadcast_to(w2c[:_BIO, :, None], (_BIO, _DIM, 16)).reshape(-1)
    b2x_flat = jnp.broadcast_to(b2c.reshape(_DIM, 1), (_DIM, 16)).reshape(-1)
    gx_flat = jnp.broadcast_to(gamma.reshape(_DIM, 1), (_DIM, 16)).reshape(-1)
    btx_flat = jnp.broadcast_to(beta.reshape(_DIM, 1), (_DIM, 16)).reshape(-1)

    out = _sc_run(idx, bio.reshape(-1), t2c_flat, w2x_flat, b2x_flat,
                  gx_flat, btx_flat)
    return out.reshape(_B, _S, _DIM)


# speedup vs baseline: 1.3286x; 1.3286x over previous
import functools

import jax
import jax.numpy as jnp
from jax import lax
from jax.experimental import pallas as pl
from jax.experimental.pallas import tpu as pltpu
from jax.experimental.pallas import tpu_sc as plsc

_B, _S = 128, 1024
_VOCAB, _DIM, _PAD, _BIO = 25, 256, 20, 5
_N = _B * _S
_T = 8192

_NW = 32
_NTOK = _N // _NW
_G = 32
_NSUB = _G // 16
_NGROUP = _NTOK // _G
_PHTOK = 1024
_GPP = _PHTOK // _G


def _fold_body(table_ref, wf_ref, wp_ref, bp_ref, bf_ref, w1_ref, w2_ref, b2_ref):
    tbl = table_ref[...]
    row = lax.broadcasted_iota(jnp.int32, tbl.shape, 0)
    tbl = jnp.where(row == _PAD, 0.0, tbl)
    wf_top = wf_ref[0:_DIM, :]
    wf_bot = wf_ref[_DIM:2 * _DIM, :]
    w1 = jnp.dot(tbl, wf_top, preferred_element_type=jnp.float32)
    w2 = jnp.dot(wp_ref[...], wf_bot, preferred_element_type=jnp.float32)
    b2 = jnp.dot(bp_ref[...], wf_bot, preferred_element_type=jnp.float32) + bf_ref[...]
    w1_ref[...] = w1 - jnp.mean(w1, axis=1, keepdims=True)
    w2_ref[...] = w2 - jnp.mean(w2, axis=1, keepdims=True)
    b2_ref[...] = b2 - jnp.mean(b2, axis=1, keepdims=True)


def _fold(table, Wp, bp, Wf, bf):
    table_pad = jnp.pad(table, ((0, 32 - _VOCAB), (0, 0)))
    wp_pad = jnp.pad(Wp, ((0, 8 - _BIO), (0, 0)))
    return pl.pallas_call(
        _fold_body,
        out_shape=(
            jax.ShapeDtypeStruct((32, _DIM), jnp.float32),
            jax.ShapeDtypeStruct((8, _DIM), jnp.float32),
            jax.ShapeDtypeStruct((1, _DIM), jnp.float32),
        ),
    )(table_pad, Wf, wp_pad, bp.reshape(1, _DIM), bf.reshape(1, _DIM))


def _tc_body(idx_ref, bio_ref, w1_ref, w2_ref, b2_ref, g_ref, bt_ref, out_ref):
    idx = idx_ref[...]
    oh = (idx == lax.broadcasted_iota(jnp.int32, (_T, 32), 1)).astype(jnp.float32)
    c = jnp.dot(oh, w1_ref[...], preferred_element_type=jnp.float32)
    c = c + jnp.dot(bio_ref[...], w2_ref[...], preferred_element_type=jnp.float32)
    c = c + b2_ref[...]
    v = jnp.mean(c * c, axis=1, keepdims=True)
    out_ref[...] = c * lax.rsqrt(v + 1e-5) * g_ref[...] + bt_ref[...]


def _sc_body(idx_hbm, bio_hbm, t2c_hbm, w2x_hbm, b2x_hbm, gx_hbm, btx_hbm,
             out_hbm,
             idx_v, bio_v, t2c_v, w2x_v, b2x_v, gx_v, btx_v, stage_v, outbuf_v,
             sem0, sem1):
    cid = lax.axis_index("c")
    sid = lax.axis_index("s")
    wid = sid * 2 + cid
    tok0 = wid * _NTOK

    pltpu.sync_copy(idx_hbm.at[pl.ds(tok0, _NTOK)], idx_v)
    pltpu.sync_copy(t2c_hbm, t2c_v)
    pltpu.sync_copy(w2x_hbm, w2x_v)
    pltpu.sync_copy(b2x_hbm, b2x_v)
    pltpu.sync_copy(gx_hbm, gx_v)
    pltpu.sync_copy(btx_hbm, btx_v)

    lane = lax.iota(jnp.int32, 16)
    sems = [sem0, sem1]

    def pair_body(p, carry):
      @pl.when(lax.rem(p, _GPP // 2) == 0)
      def _stage_bio():
          ph = lax.div(p, _GPP // 2)
          pltpu.sync_copy(
              bio_hbm.at[pl.ds((tok0 + ph * _PHTOK) * 8, _PHTOK * 8)], bio_v)

      for h in range(2):
        g = p * 2 + h
        base = g * _G
        lbase = lax.rem(g, _GPP) * _G
        rowaddr = []
        bvecs = []
        for s in range(_NSUB):
            idxv = idx_v[pl.ds(base + s * 16, 16)]
            rowaddr.append(idxv * 257)
            baddr = (lbase + s * 16) * 8 + lane * 8
            bvecs.append([plsc.load_gather(bio_v, [baddr + k]) for k in range(_BIO)])

        zero = jnp.zeros((16,), jnp.float32)

        def p1(i, accs):
            new = list(accs)
            for u in range(2):
                d = i + u
                w = [w2x_v[pl.ds(k * (_DIM * 16) + d * 16, 16)] for k in range(_BIO)]
                b2 = b2x_v[pl.ds(d * 16, 16)]
                for s in range(_NSUB):
                    g0 = plsc.load_gather(t2c_v, [rowaddr[s] + d]) + b2
                    m01 = bvecs[s][0] * w[0] + bvecs[s][1] * w[1]
                    m23 = bvecs[s][2] * w[2] + bvecs[s][3] * w[3]
                    e = (g0 + bvecs[s][4] * w[4]) + (m01 + m23)
                    stage_v[pl.ds(d * _G + s * 16, 16)] = e
                    j = u * _NSUB + s
                    new[j] = new[j] + e * e
            return tuple(new)

        accs = plsc.parallel_loop(
            0, _DIM, 2, unroll=2,
            carry=tuple(zero for _ in range(2 * _NSUB)))(p1)

        invs = []
        for s in range(_NSUB):
            x = (accs[s] + accs[_NSUB + s]) * (1.0 / _DIM) + 1e-5
            i32 = plsc.bitcast(x, jnp.int32)
            y = plsc.bitcast(jnp.int32(0x5F3759DF) - (i32 >> 1), jnp.float32)
            for _ in range(3):
                y = y * (1.5 - 0.5 * x * y * y)
            invs.append(y)

        rowidx = [h * _G + lane + s * 16 for s in range(_NSUB)]

        @pl.when(p > 0)
        def _wait_prev():
            pltpu.make_async_copy(
                outbuf_v.at[pl.ds(h * _G, _G), 0:_DIM],
                out_hbm.at[pl.ds(tok0 + (g - 2) * _G, _G), :],
                sems[h]).wait()

        def p2(d, c2):
            gsp = gx_v[pl.ds(d * 16, 16)]
            bsp = btx_v[pl.ds(d * 16, 16)]
            dcol = jnp.full((16,), d, jnp.int32)
            for s in range(_NSUB):
                e = stage_v[pl.ds(d * _G + s * 16, 16)]
                plsc.store_scatter(outbuf_v, [rowidx[s], dcol],
                                   (e * invs[s]) * gsp + bsp)
            return c2

        plsc.parallel_loop(0, _DIM, 1, unroll=8, carry=jnp.int32(0))(p2)
        pltpu.make_async_copy(outbuf_v.at[pl.ds(h * _G, _G), 0:_DIM],
                              out_hbm.at[pl.ds(tok0 + base, _G), :],
                              sems[h]).start()
      return carry

    lax.fori_loop(0, _NGROUP // 2, pair_body, 0)
    for h in range(2):
        g = _NGROUP - 2 + h
        pltpu.make_async_copy(outbuf_v.at[pl.ds(h * _G, _G), 0:_DIM],
                              out_hbm.at[pl.ds(tok0 + g * _G, _G), :],
                              sems[h]).wait()


def _sc_run(idx, bio8_flat, t2c_flat, w2x_flat, b2x_flat, gx_flat, btx_flat):
    mesh = plsc.VectorSubcoreMesh(core_axis_name="c", subcore_axis_name="s")
    f = functools.partial(
        pl.kernel, _sc_body,
        out_type=jax.ShapeDtypeStruct((_N, _DIM), jnp.float32),
        mesh=mesh,
        scratch_types=[
            pltpu.VMEM((_NTOK,), jnp.int32),
            pltpu.VMEM((_PHTOK * 8,), jnp.float32),
            pltpu.VMEM((32 * 257,), jnp.float32),
            pltpu.VMEM((_BIO * _DIM * 16,), jnp.float32),
            pltpu.VMEM((_DIM * 16,), jnp.float32),
            pltpu.VMEM((_DIM * 16,), jnp.float32),
            pltpu.VMEM((_DIM * 16,), jnp.float32),
            pltpu.VMEM((_DIM * _G,), jnp.float32),
            pltpu.VMEM((2 * _G, 257), jnp.float32),
            pltpu.SemaphoreType.DMA,
            pltpu.SemaphoreType.DMA,
        ],
        compiler_params=pltpu.CompilerParams(needs_layout_passes=False),
    )()
    return f(idx, bio8_flat, t2c_flat, w2x_flat, b2x_flat, gx_flat, btx_flat)


def kernel(aa_indices, biochem_features, table, Wp, bp, Wf, bf, gamma, beta):
    idx = aa_indices.astype(jnp.int32).reshape(_N)
    bio = jnp.pad(biochem_features.reshape(_N, _BIO), ((0, 0), (0, 8 - _BIO)))
    w1c, w2c, b2c = _fold(table, Wp, bp, Wf, bf)

    t2c_flat = jnp.pad(w1c, ((0, 0), (0, 1))).reshape(-1)
    w2x_flat = jnp.broadcast_to(w2c[:_BIO, :, None], (_BIO, _DIM, 16)).reshape(-1)
    b2x_flat = jnp.broadcast_to(b2c.reshape(_DIM, 1), (_DIM, 16)).reshape(-1)
    gx_flat = jnp.broadcast_to(gamma.reshape(_DIM, 1), (_DIM, 16)).reshape(-1)
    btx_flat = jnp.broadcast_to(beta.reshape(_DIM, 1), (_DIM, 16)).reshape(-1)

    out = _sc_run(idx, bio.reshape(-1), t2c_flat, w2x_flat, b2x_flat,
                  gx_flat, btx_flat)
    return out.reshape(_B, _S, _DIM)


# speedup vs baseline: 1.3496x; 1.0158x over previous
import functools

import jax
import jax.numpy as jnp
from jax import lax
from jax.experimental import pallas as pl
from jax.experimental.pallas import tpu as pltpu
from jax.experimental.pallas import tpu_sc as plsc

_B, _S = 128, 1024
_VOCAB, _DIM, _PAD, _BIO = 25, 256, 20, 5
_N = _B * _S
_T = 8192

_NW = 32
_NTOK = _N // _NW
_G = 32
_NSUB = _G // 16
_NGROUP = _NTOK // _G
_PHTOK = 1024
_GPP = _PHTOK // _G


def _fold_body(table_ref, wf_ref, wp_ref, bp_ref, bf_ref, w1_ref, w2_ref, b2_ref):
    tbl = table_ref[...]
    row = lax.broadcasted_iota(jnp.int32, tbl.shape, 0)
    tbl = jnp.where(row == _PAD, 0.0, tbl)
    wf_top = wf_ref[0:_DIM, :]
    wf_bot = wf_ref[_DIM:2 * _DIM, :]
    w1 = jnp.dot(tbl, wf_top, preferred_element_type=jnp.float32)
    w2 = jnp.dot(wp_ref[...], wf_bot, preferred_element_type=jnp.float32)
    b2 = jnp.dot(bp_ref[...], wf_bot, preferred_element_type=jnp.float32) + bf_ref[...]
    w1_ref[...] = w1 - jnp.mean(w1, axis=1, keepdims=True)
    w2_ref[...] = w2 - jnp.mean(w2, axis=1, keepdims=True)
    b2_ref[...] = b2 - jnp.mean(b2, axis=1, keepdims=True)


def _fold(table, Wp, bp, Wf, bf):
    table_pad = jnp.pad(table, ((0, 32 - _VOCAB), (0, 0)))
    wp_pad = jnp.pad(Wp, ((0, 8 - _BIO), (0, 0)))
    return pl.pallas_call(
        _fold_body,
        out_shape=(
            jax.ShapeDtypeStruct((32, _DIM), jnp.float32),
            jax.ShapeDtypeStruct((8, _DIM), jnp.float32),
            jax.ShapeDtypeStruct((1, _DIM), jnp.float32),
        ),
    )(table_pad, Wf, wp_pad, bp.reshape(1, _DIM), bf.reshape(1, _DIM))


def _tc_body(idx_ref, bio_ref, w1_ref, w2_ref, b2_ref, g_ref, bt_ref, out_ref):
    idx = idx_ref[...]
    oh = (idx == lax.broadcasted_iota(jnp.int32, (_T, 32), 1)).astype(jnp.float32)
    c = jnp.dot(oh, w1_ref[...], preferred_element_type=jnp.float32)
    c = c + jnp.dot(bio_ref[...], w2_ref[...], preferred_element_type=jnp.float32)
    c = c + b2_ref[...]
    v = jnp.mean(c * c, axis=1, keepdims=True)
    out_ref[...] = c * lax.rsqrt(v + 1e-5) * g_ref[...] + bt_ref[...]


def _sc_body(idx_hbm, bio_hbm, t2c_hbm, w2x_hbm, b2x_hbm, gx_hbm, btx_hbm,
             out_hbm,
             idx_v, bio_v, t2c_v, w2x_v, b2x_v, gx_v, btx_v, stage_v, outbuf_v,
             sem0, sem1):
    cid = lax.axis_index("c")
    sid = lax.axis_index("s")
    wid = sid * 2 + cid
    tok0 = wid * _NTOK

    pltpu.sync_copy(idx_hbm.at[pl.ds(tok0, _NTOK)], idx_v)
    pltpu.sync_copy(t2c_hbm, t2c_v)
    pltpu.sync_copy(w2x_hbm, w2x_v)
    pltpu.sync_copy(b2x_hbm, b2x_v)
    pltpu.sync_copy(gx_hbm, gx_v)
    pltpu.sync_copy(btx_hbm, btx_v)

    lane = lax.iota(jnp.int32, 16)
    sems = [sem0, sem1]

    def pair_body(p, carry):
      @pl.when(lax.rem(p, _GPP // 2) == 0)
      def _stage_bio():
          ph = lax.div(p, _GPP // 2)
          pltpu.sync_copy(
              bio_hbm.at[pl.ds((tok0 + ph * _PHTOK) * 8, _PHTOK * 8)], bio_v)

      for h in range(2):
        g = p * 2 + h
        base = g * _G
        lbase = lax.rem(g, _GPP) * _G
        rowaddr = []
        bvecs = []
        for s in range(_NSUB):
            idxv = idx_v[pl.ds(base + s * 16, 16)]
            rowaddr.append(idxv * 257)
            baddr = (lbase + s * 16) * 8 + lane * 8
            bvecs.append([plsc.load_gather(bio_v, [baddr + k]) for k in range(_BIO)])

        zero = jnp.zeros((16,), jnp.float32)

        def p1(i, accs):
            new = list(accs)
            for u in range(2):
                d = i + u
                w = [w2x_v[pl.ds(k * (_DIM * 16) + d * 16, 16)] for k in range(_BIO)]
                b2 = b2x_v[pl.ds(d * 16, 16)]
                for s in range(_NSUB):
                    g0 = plsc.load_gather(t2c_v, [rowaddr[s] + d]) + b2
                    m01 = bvecs[s][0] * w[0] + bvecs[s][1] * w[1]
                    m23 = bvecs[s][2] * w[2] + bvecs[s][3] * w[3]
                    e = (g0 + bvecs[s][4] * w[4]) + (m01 + m23)
                    stage_v[pl.ds(d * _G + s * 16, 16)] = e
                    j = u * _NSUB + s
                    new[j] = new[j] + e * e
            return tuple(new)

        accs = plsc.parallel_loop(
            0, _DIM, 2, unroll=2,
            carry=tuple(zero for _ in range(2 * _NSUB)))(p1)

        invs = []
        for s in range(_NSUB):
            x = (accs[s] + accs[_NSUB + s]) * (1.0 / _DIM) + 1e-5
            i32 = plsc.bitcast(x, jnp.int32)
            y = plsc.bitcast(jnp.int32(0x5F3759DF) - (i32 >> 1), jnp.float32)
            for _ in range(3):
                y = y * (1.5 - 0.5 * x * y * y)
            invs.append(y)

        rowidx = [h * _G + lane + s * 16 for s in range(_NSUB)]

        @pl.when(p > 0)
        def _wait_prev():
            pltpu.make_async_copy(
                outbuf_v.at[pl.ds(h * _G, _G), 0:_DIM],
                out_hbm.at[pl.ds(tok0 + (g - 2) * _G, _G), :],
                sems[h]).wait()

        def p2(d, c2):
            gsp = gx_v[pl.ds(d * 16, 16)]
            bsp = btx_v[pl.ds(d * 16, 16)]
            dcol = jnp.full((16,), d, jnp.int32)
            for s in range(_NSUB):
                e = stage_v[pl.ds(d * _G + s * 16, 16)]
                plsc.store_scatter(outbuf_v, [rowidx[s], dcol],
                                   (e * invs[s]) * gsp + bsp)
            return c2

        plsc.parallel_loop(0, _DIM, 1, unroll=4, carry=jnp.int32(0))(p2)
        pltpu.make_async_copy(outbuf_v.at[pl.ds(h * _G, _G), 0:_DIM],
                              out_hbm.at[pl.ds(tok0 + base, _G), :],
                              sems[h]).start()
      return carry

    lax.fori_loop(0, _NGROUP // 2, pair_body, 0)
    for h in range(2):
        g = _NGROUP - 2 + h
        pltpu.make_async_copy(outbuf_v.at[pl.ds(h * _G, _G), 0:_DIM],
                              out_hbm.at[pl.ds(tok0 + g * _G, _G), :],
                              sems[h]).wait()


def _sc_run(idx, bio8_flat, t2c_flat, w2x_flat, b2x_flat, gx_flat, btx_flat):
    mesh = plsc.VectorSubcoreMesh(core_axis_name="c", subcore_axis_name="s")
    f = functools.partial(
        pl.kernel, _sc_body,
        out_type=jax.ShapeDtypeStruct((_N, _DIM), jnp.float32),
        mesh=mesh,
        scratch_types=[
            pltpu.VMEM((_NTOK,), jnp.int32),
            pltpu.VMEM((_PHTOK * 8,), jnp.float32),
            pltpu.VMEM((32 * 257,), jnp.float32),
            pltpu.VMEM((_BIO * _DIM * 16,), jnp.float32),
            pltpu.VMEM((_DIM * 16,), jnp.float32),
            pltpu.VMEM((_DIM * 16,), jnp.float32),
            pltpu.VMEM((_DIM * 16,), jnp.float32),
            pltpu.VMEM((_DIM * _G,), jnp.float32),
            pltpu.VMEM((2 * _G, 257), jnp.float32),
            pltpu.SemaphoreType.DMA,
            pltpu.SemaphoreType.DMA,
        ],
        compiler_params=pltpu.CompilerParams(needs_layout_passes=False),
    )()
    return f(idx, bio8_flat, t2c_flat, w2x_flat, b2x_flat, gx_flat, btx_flat)


def kernel(aa_indices, biochem_features, table, Wp, bp, Wf, bf, gamma, beta):
    idx = aa_indices.astype(jnp.int32).reshape(_N)
    bio = jnp.pad(biochem_features.reshape(_N, _BIO), ((0, 0), (0, 8 - _BIO)))
    w1c, w2c, b2c = _fold(table, Wp, bp, Wf, bf)

    t2c_flat = jnp.pad(w1c, ((0, 0), (0, 1))).reshape(-1)
    w2x_flat = jnp.broadcast_to(w2c[:_BIO, :, None], (_BIO, _DIM, 16)).reshape(-1)
    b2x_flat = jnp.broadcast_to(b2c.reshape(_DIM, 1), (_DIM, 16)).reshape(-1)
    gx_flat = jnp.broadcast_to(gamma.reshape(_DIM, 1), (_DIM, 16)).reshape(-1)
    btx_flat = jnp.broadcast_to(beta.reshape(_DIM, 1), (_DIM, 16)).reshape(-1)

    out = _sc_run(idx, bio.reshape(-1), t2c_flat, w2x_flat, b2x_flat,
                  gx_flat, btx_flat)
    return out.reshape(_B, _S, _DIM)


# speedup vs baseline: 1.3658x; 1.0120x over previous
import functools

import jax
import jax.numpy as jnp
from jax import lax
from jax.experimental import pallas as pl
from jax.experimental.pallas import tpu as pltpu
from jax.experimental.pallas import tpu_sc as plsc

_B, _S = 128, 1024
_VOCAB, _DIM, _PAD, _BIO = 25, 256, 20, 5
_N = _B * _S
_T = 8192

_NW = 32
_NTOK = _N // _NW
_G = 32
_NSUB = _G // 16
_NGROUP = _NTOK // _G
_PHTOK = 1024
_GPP = _PHTOK // _G


def _fold_body(table_ref, wf_ref, wp_ref, bp_ref, bf_ref, w1_ref, w2_ref, b2_ref):
    tbl = table_ref[...]
    row = lax.broadcasted_iota(jnp.int32, tbl.shape, 0)
    tbl = jnp.where(row == _PAD, 0.0, tbl)
    wf_top = wf_ref[0:_DIM, :]
    wf_bot = wf_ref[_DIM:2 * _DIM, :]
    w1 = jnp.dot(tbl, wf_top, preferred_element_type=jnp.float32)
    w2 = jnp.dot(wp_ref[...], wf_bot, preferred_element_type=jnp.float32)
    b2 = jnp.dot(bp_ref[...], wf_bot, preferred_element_type=jnp.float32) + bf_ref[...]
    w1_ref[...] = w1 - jnp.mean(w1, axis=1, keepdims=True)
    w2_ref[...] = w2 - jnp.mean(w2, axis=1, keepdims=True)
    b2_ref[...] = b2 - jnp.mean(b2, axis=1, keepdims=True)


def _fold(table, Wp, bp, Wf, bf):
    table_pad = jnp.pad(table, ((0, 32 - _VOCAB), (0, 0)))
    wp_pad = jnp.pad(Wp, ((0, 8 - _BIO), (0, 0)))
    return pl.pallas_call(
        _fold_body,
        out_shape=(
            jax.ShapeDtypeStruct((32, _DIM), jnp.float32),
            jax.ShapeDtypeStruct((8, _DIM), jnp.float32),
            jax.ShapeDtypeStruct((1, _DIM), jnp.float32),
        ),
    )(table_pad, Wf, wp_pad, bp.reshape(1, _DIM), bf.reshape(1, _DIM))


def _tc_body(idx_ref, bio_ref, w1_ref, w2_ref, b2_ref, g_ref, bt_ref, out_ref):
    idx = idx_ref[...]
    oh = (idx == lax.broadcasted_iota(jnp.int32, (_T, 32), 1)).astype(jnp.float32)
    c = jnp.dot(oh, w1_ref[...], preferred_element_type=jnp.float32)
    c = c + jnp.dot(bio_ref[...], w2_ref[...], preferred_element_type=jnp.float32)
    c = c + b2_ref[...]
    v = jnp.mean(c * c, axis=1, keepdims=True)
    out_ref[...] = c * lax.rsqrt(v + 1e-5) * g_ref[...] + bt_ref[...]


def _sc_body(idx_hbm, bio_hbm, t2c_hbm, w2x_hbm, b2x_hbm, gx_hbm, btx_hbm,
             out_hbm,
             idx_v, bio_v, t2c_v, w2x_v, b2x_v, gx_v, btx_v, stage_v, outbuf_v,
             sem0, sem1):
    cid = lax.axis_index("c")
    sid = lax.axis_index("s")
    wid = sid * 2 + cid
    tok0 = wid * _NTOK

    pltpu.sync_copy(idx_hbm.at[pl.ds(tok0, _NTOK)], idx_v)
    pltpu.sync_copy(t2c_hbm, t2c_v)
    pltpu.sync_copy(w2x_hbm, w2x_v)
    pltpu.sync_copy(b2x_hbm, b2x_v)
    pltpu.sync_copy(gx_hbm, gx_v)
    pltpu.sync_copy(btx_hbm, btx_v)

    lane = lax.iota(jnp.int32, 16)
    sems = [sem0, sem1]

    def pair_body(p, carry):
      @pl.when(lax.rem(p, _GPP // 2) == 0)
      def _stage_bio():
          ph = lax.div(p, _GPP // 2)
          pltpu.sync_copy(
              bio_hbm.at[pl.ds((tok0 + ph * _PHTOK) * 8, _PHTOK * 8)], bio_v)

      for h in range(2):
        g = p * 2 + h
        base = g * _G
        lbase = lax.rem(g, _GPP) * _G
        rowaddr = []
        bvecs = []
        for s in range(_NSUB):
            idxv = idx_v[pl.ds(base + s * 16, 16)]
            rowaddr.append(idxv * 257)
            baddr = (lbase + s * 16) * 8 + lane * 8
            bvecs.append([plsc.load_gather(bio_v, [baddr + k]) for k in range(_BIO)])

        zero = jnp.zeros((16,), jnp.float32)

        def p1(i, accs):
            new = list(accs)
            for u in range(2):
                d = i + u
                w = [w2x_v[pl.ds(k * (_DIM * 16) + d * 16, 16)] for k in range(_BIO)]
                b2 = b2x_v[pl.ds(d * 16, 16)]
                for s in range(_NSUB):
                    g0 = plsc.load_gather(t2c_v, [rowaddr[s] + d]) + b2
                    m01 = bvecs[s][0] * w[0] + bvecs[s][1] * w[1]
                    m23 = bvecs[s][2] * w[2] + bvecs[s][3] * w[3]
                    e = (g0 + bvecs[s][4] * w[4]) + (m01 + m23)
                    stage_v[pl.ds(d * _G + s * 16, 16)] = e
                    j = u * _NSUB + s
                    new[j] = new[j] + e * e
            return tuple(new)

        accs = plsc.parallel_loop(
            0, _DIM, 2, unroll=2,
            carry=tuple(zero for _ in range(2 * _NSUB)))(p1)

        invs = []
        for s in range(_NSUB):
            x = (accs[s] + accs[_NSUB + s]) * (1.0 / _DIM) + 1e-5
            i32 = plsc.bitcast(x, jnp.int32)
            y = plsc.bitcast(jnp.int32(0x5F3759DF) - (i32 >> 1), jnp.float32)
            for _ in range(3):
                y = y * (1.5 - 0.5 * x * y * y)
            invs.append(y)

        rowidx = [h * _G + lane + s * 16 for s in range(_NSUB)]

        @pl.when(p > 0)
        def _wait_prev():
            pltpu.make_async_copy(
                outbuf_v.at[pl.ds(h * _G, _G), 0:_DIM],
                out_hbm.at[pl.ds(tok0 + (g - 2) * _G, _G), :],
                sems[h]).wait()

        def p2(d, c2):
            gsp = gx_v[pl.ds(d * 16, 16)]
            bsp = btx_v[pl.ds(d * 16, 16)]
            dcol = jnp.full((16,), d, jnp.int32)
            for s in range(_NSUB):
                e = stage_v[pl.ds(d * _G + s * 16, 16)]
                plsc.store_scatter(outbuf_v, [rowidx[s], dcol],
                                   (e * invs[s]) * gsp + bsp)
            return c2

        plsc.parallel_loop(0, _DIM, 1, unroll=2, carry=jnp.int32(0))(p2)
        pltpu.make_async_copy(outbuf_v.at[pl.ds(h * _G, _G), 0:_DIM],
                              out_hbm.at[pl.ds(tok0 + base, _G), :],
                              sems[h]).start()
      return carry

    lax.fori_loop(0, _NGROUP // 2, pair_body, 0)
    for h in range(2):
        g = _NGROUP - 2 + h
        pltpu.make_async_copy(outbuf_v.at[pl.ds(h * _G, _G), 0:_DIM],
                              out_hbm.at[pl.ds(tok0 + g * _G, _G), :],
                              sems[h]).wait()


def _sc_run(idx, bio8_flat, t2c_flat, w2x_flat, b2x_flat, gx_flat, btx_flat):
    mesh = plsc.VectorSubcoreMesh(core_axis_name="c", subcore_axis_name="s")
    f = functools.partial(
        pl.kernel, _sc_body,
        out_type=jax.ShapeDtypeStruct((_N, _DIM), jnp.float32),
        mesh=mesh,
        scratch_types=[
            pltpu.VMEM((_NTOK,), jnp.int32),
            pltpu.VMEM((_PHTOK * 8,), jnp.float32),
            pltpu.VMEM((32 * 257,), jnp.float32),
            pltpu.VMEM((_BIO * _DIM * 16,), jnp.float32),
            pltpu.VMEM((_DIM * 16,), jnp.float32),
            pltpu.VMEM((_DIM * 16,), jnp.float32),
            pltpu.VMEM((_DIM * 16,), jnp.float32),
            pltpu.VMEM((_DIM * _G,), jnp.float32),
            pltpu.VMEM((2 * _G, 257), jnp.float32),
            pltpu.SemaphoreType.DMA,
            pltpu.SemaphoreType.DMA,
        ],
        compiler_params=pltpu.CompilerParams(needs_layout_passes=False),
    )()
    return f(idx, bio8_flat, t2c_flat, w2x_flat, b2x_flat, gx_flat, btx_flat)


def kernel(aa_indices, biochem_features, table, Wp, bp, Wf, bf, gamma, beta):
    idx = aa_indices.astype(jnp.int32).reshape(_N)
    bio = jnp.pad(biochem_features.reshape(_N, _BIO), ((0, 0), (0, 8 - _BIO)))
    w1c, w2c, b2c = _fold(table, Wp, bp, Wf, bf)

    t2c_flat = jnp.pad(w1c, ((0, 0), (0, 1))).reshape(-1)
    w2x_flat = jnp.broadcast_to(w2c[:_BIO, :, None], (_BIO, _DIM, 16)).reshape(-1)
    b2x_flat = jnp.broadcast_to(b2c.reshape(_DIM, 1), (_DIM, 16)).reshape(-1)
    gx_flat = jnp.broadcast_to(gamma.reshape(_DIM, 1), (_DIM, 16)).reshape(-1)
    btx_flat = jnp.broadcast_to(beta.reshape(_DIM, 1), (_DIM, 16)).reshape(-1)

    out = _sc_run(idx, bio.reshape(-1), t2c_flat, w2x_flat, b2x_flat,
                  gx_flat, btx_flat)
    return out.reshape(_B, _S, _DIM)
